# Initial kernel scaffold; baseline (speedup 1.0000x reference)
import jax
import jax.numpy as jnp
from jax.experimental import pallas as pl


def kernel(x, neigh_orders, pool_neigh_orders, W1, b1, g1, be1, W2, b2, g2, be2):
    raise NotImplementedError("write your pallas kernel here")


# speedup vs baseline: 3.0461x; 3.0461x over previous
import functools

import jax
import jax.numpy as jnp
from jax import lax
from jax.experimental import pallas as pl
from jax.experimental.pallas import tpu as pltpu
from jax.experimental.pallas import tpu_sc as plsc

C = 32
V_IN = 163842
V_OUT = 40962
B = 2
EPS = 1e-5
NEG = 0.2

R = 1024
VP = 43008
NB = VP // R
NP = B * VP
NPB = NP // R
CH = 128
NCH = NP // CH
NW = 32
CPW = NCH // NW
NBO = (V_OUT + R - 1) // R


def _sc_mesh():
    return plsc.VectorSubcoreMesh(core_axis_name="c", subcore_axis_name="s")


def _gather7_body(table_hbm, idxc_hbm, out_hbm, idx_v, acc_v, sem):
    wid = lax.axis_index("s") * 2 + lax.axis_index("c")

    zv = jnp.zeros((16,), jnp.float32)

    def zrow(r, carry):
        acc_v[r, pl.ds(0, 16)] = zv
        acc_v[r, pl.ds(16, 16)] = zv
        return carry

    def chunk(c, carry):
        k = wid * CPW + c
        pltpu.sync_copy(idxc_hbm.at[k], idx_v)
        lax.fori_loop(0, CH, zrow, 0)
        cps = [
            pltpu.async_copy(table_hbm.at[idx_v.at[j]], acc_v, sem, add=True)
            for j in range(7)
        ]
        for cp in cps:
            cp.wait()
        pltpu.sync_copy(acc_v, out_hbm.at[pl.ds(k * CH, CH)])
        return carry

    lax.fori_loop(0, CPW, chunk, 0)


def _gather7(table, idx_chunks):
    f = functools.partial(
        pl.kernel,
        out_type=jax.ShapeDtypeStruct((NP, C), jnp.float32),
        mesh=_sc_mesh(),
        scratch_types=[
            pltpu.VMEM((7, CH), jnp.int32),
            pltpu.VMEM((CH, C), jnp.float32),
            pltpu.SemaphoreType.DMA,
        ],
        compiler_params=pltpu.CompilerParams(use_tc_tiling_on_sc=False),
    )(_gather7_body)
    return f(table, idx_chunks)


def _mm_body(h_ref, w_ref, p_ref):
    y = jnp.dot(h_ref[...], w_ref[...], preferred_element_type=jnp.float32,
                precision=lax.Precision.HIGHEST)
    p_ref[...] = y.reshape(R, 7, C)


def _mm(h, w):
    return pl.pallas_call(
        _mm_body,
        grid=(NPB,),
        in_specs=[
            pl.BlockSpec((R, C), lambda i: (i, 0)),
            pl.BlockSpec((C, 7 * C), lambda i: (0, 0)),
        ],
        out_specs=pl.BlockSpec((R, 7, C), lambda i: (i, 0, 0)),
        out_shape=jax.ShapeDtypeStruct((NP, 7, C), jnp.float32),
    )(h, w)


def _stats_body(c_ref, o_ref):
    i = pl.program_id(1)
    x = c_ref[...]
    rows = lax.broadcasted_iota(jnp.int32, (R, 1), 0) + i * R
    xm = jnp.where(rows < V_OUT, x, 0.0)
    s1 = jnp.sum(xm, axis=0)
    s2 = jnp.sum(xm * xm, axis=0)
    blk = jnp.concatenate(
        [
            jnp.pad(s1, (0, 128 - C))[None, :],
            jnp.pad(s2, (0, 128 - C))[None, :],
            jnp.zeros((6, 128), jnp.float32),
        ],
        axis=0,
    )[None]

    @pl.when(i == 0)
    def _():
        o_ref[...] = blk

    @pl.when(i > 0)
    def _():
        o_ref[...] += blk


def _stats(c):
    return pl.pallas_call(
        _stats_body,
        grid=(B, NB),
        in_specs=[pl.BlockSpec((R, C), lambda b, i: (b * NB + i, 0))],
        out_specs=pl.BlockSpec((1, 8, 128), lambda b, i: (b, 0, 0)),
        out_shape=jax.ShapeDtypeStruct((B, 8, 128), jnp.float32),
    )(c)


def _bn_act(c_ref, s_ref, gb_ref):
    s = s_ref[0]
    mean = s[0:1, :C] / V_OUT
    var = s[1:2, :C] / V_OUT - mean * mean
    scale = gb_ref[0:1, :C] * lax.rsqrt(var + EPS)
    shift = gb_ref[1:2, :C] - mean * scale
    h = c_ref[...] * scale + shift
    return jnp.where(h >= 0, h, NEG * h)


def _bnmm_body(c_ref, s_ref, gb_ref, w_ref, p_ref):
    h = _bn_act(c_ref, s_ref, gb_ref)
    y = jnp.dot(h, w_ref[...], preferred_element_type=jnp.float32,
                precision=lax.Precision.HIGHEST)
    p_ref[...] = y.reshape(R, 7, C)


def _bnmm(c, s, gb, w):
    return pl.pallas_call(
        _bnmm_body,
        grid=(B, NB),
        in_specs=[
            pl.BlockSpec((R, C), lambda b, i: (b * NB + i, 0)),
            pl.BlockSpec((1, 8, 128), lambda b, i: (b, 0, 0)),
            pl.BlockSpec((8, 128), lambda b, i: (0, 0)),
            pl.BlockSpec((C, 7 * C), lambda b, i: (0, 0)),
        ],
        out_specs=pl.BlockSpec((R, 7, C), lambda b, i: (b * NB + i, 0, 0)),
        out_shape=jax.ShapeDtypeStruct((NP, 7, C), jnp.float32),
    )(c, s, gb, w)


def _final_body(c_ref, s_ref, gb_ref, o_ref):
    h = _bn_act(c_ref, s_ref, gb_ref)
    o_ref[...] = h.T[None]


def _final(c, s, gb):
    return pl.pallas_call(
        _final_body,
        grid=(B, NBO),
        in_specs=[
            pl.BlockSpec((R, C), lambda b, i: (b * NB + i, 0)),
            pl.BlockSpec((1, 8, 128), lambda b, i: (b, 0, 0)),
            pl.BlockSpec((8, 128), lambda b, i: (0, 0)),
        ],
        out_specs=pl.BlockSpec((1, C, R), lambda b, i: (b, 0, i)),
        out_shape=jax.ShapeDtypeStruct((B, C, V_OUT), jnp.float32),
    )(c, s, gb)


def _pack_gb(g, be):
    return jnp.concatenate(
        [
            jnp.pad(g, (0, 128 - C))[None, :],
            jnp.pad(be, (0, 128 - C))[None, :],
            jnp.zeros((6, 128), jnp.float32),
        ],
        axis=0,
    )


def kernel(x, neigh_orders, pool_neigh_orders, W1, b1, g1, be1, W2, b2, g2, be2):
    xt = x.transpose(0, 2, 1).reshape(B * V_IN, C)

    boff = jnp.arange(B, dtype=jnp.int32)
    ar7 = jnp.arange(7, dtype=jnp.int32)

    pop = jnp.pad(pool_neigh_orders.reshape(V_OUT, 7), ((0, VP - V_OUT), (0, 0)))
    ip = pop[None] + (boff * V_IN)[:, None, None]
    ip = ip.reshape(NCH, CH, 7).transpose(0, 2, 1)

    nop = jnp.pad(neigh_orders.reshape(V_OUT, 7), ((0, VP - V_OUT), (0, 0)))
    ic = (nop[None] + (boff * VP)[:, None, None]) * 7 + ar7[None, None, :]
    ic = ic.reshape(NCH, CH, 7).transpose(0, 2, 1)

    w1 = W1.reshape(7, C, C).transpose(1, 0, 2).reshape(C, 7 * C) / 7.0
    w2 = W2.reshape(7, C, C).transpose(1, 0, 2).reshape(C, 7 * C)
    gb1 = _pack_gb(g1, be1)
    gb2 = _pack_gb(g2, be2)

    h0 = _gather7(xt, ip)
    p1 = _mm(h0, w1).reshape(NP * 7, C)
    c1 = _gather7(p1, ic)
    s1 = _stats(c1)
    p2 = _bnmm(c1, s1, gb1, w2).reshape(NP * 7, C)
    c2 = _gather7(p2, ic)
    s2 = _stats(c2)
    return _final(c2, s2, gb2)
